# baseline (speedup 1.0000x reference)
import math

import numpy as np
import jax
import jax.numpy as jnp
from jax.experimental import pallas as pl
from jax.experimental.pallas import tpu as pltpu

_N = 160
_RCR = 5.1
_RCA = 3.5
_ETA_R = 19.7
_SHF_R = (0.8, 1.06875, 1.3375, 1.60625, 1.875, 2.14375, 2.4125, 2.68125,
          2.95, 3.21875, 3.4875, 3.75625, 4.025, 4.29375, 4.5625, 4.83125)
_NCOLS = 7 * 16 + 28 * 8 * 8
_B = 32
_STEPS = _N // _B
_PI = math.pi
_SCALE = 1.0 / (_N * _NCOLS)

_ACEN = 1.98125

_Q1_A = -25.07469501669646
_Q1 = ((-2.316327802344783, 1.5333792477612849),
       (-1.4230545414215414, 1.0002167869105418),
       (-0.36580345570448514, 0.5132197102364305),
       (0.48813884994230666, 0.2426895429205808),
       (-0.8137611968728693, -0.4974492128018933))
_Q2_A = -0.00017303569984151917
_Q2 = ((-7.723098940441944, 14.915935134292987),
       (-7.2991807539960405, 13.354248559476355),
       (-6.652481993284735, 11.198049856411668),
       (-5.809573689410307, 8.804768403329167),
       (-4.720882331809866, 6.370811559783335),
       (-3.1307684518758734, 3.8783087973282906),
       (-1.7458215437999687, 1.2850849853111153),
       (-0.6600098398625358, 0.3599938737112747),
       (0.04629375728144155, 0.06739772475157971),
       (-3.789796635845545, -0.5663128365514963))


def _prodpoly(A, quads, x):
    b0, c0 = quads[0]
    acc = (np.float32(A) * x + np.float32(A * b0)) * x + np.float32(A * c0)
    for b, c in quads[1:]:
        acc = acc * ((x + np.float32(b)) * x + np.float32(c))
    return acc


_NL = 128


def _aev_kernel(post_ref, posc_ref, out_ref):
    step = pl.program_id(0)
    base = step * _B
    f32 = jnp.float32

    px = post_ref[0:1, :]
    py = post_ref[1:2, :]
    pz = post_ref[2:3, :]
    cblk = posc_ref[pl.ds(base, _B), :]
    cx = cblk[:, 0:1]
    cy = cblk[:, 1:2]
    cz = cblk[:, 2:3]

    dx = px - cx
    dy = py - cy
    dz = pz - cz
    d2 = dx * dx + dy * dy + dz * dz
    valid = d2 > 1e-12
    dij = jnp.where(valid, jnp.sqrt(jnp.where(valid, d2, 1.0)), 0.0)
    rinv = jnp.where(valid, 1.0 / jnp.where(valid, dij, 1.0), 0.0)

    jidx = jax.lax.broadcasted_iota(jnp.int32, (_B, _N), 1)
    cidx = jax.lax.broadcasted_iota(jnp.int32, (_B, _N), 0) + base
    ne_f = (jidx != cidx).astype(f32)

    fc_r = jnp.where(dij <= _RCR, 0.5 * jnp.cos(_PI / _RCR * dij) + 0.5, 0.0)
    fc_r = fc_r * ne_f
    racc = jnp.zeros((_B, _N), f32)
    for s in _SHF_R:
        racc += jnp.exp(-_ETA_R * (dij - s) ** 2)
    radial_part = jnp.sum(0.25 * racc * fc_r)

    fcj = jnp.where(dij <= _RCA, 0.5 * jnp.cos(_PI / _RCA * dij) + 0.5, 0.0)
    fcj = fcj * ne_f
    ux = dx * rinv
    uy = dy * rinv
    uz = dz * rinv
    g = jnp.minimum(0.5 * dij, 0.5 * _RCA) - 0.5 * _ACEN

    def pair_block(rs, cs):
        u = (ux[:, rs, None] * ux[:, None, cs]
             + uy[:, rs, None] * uy[:, None, cs]
             + uz[:, rs, None] * uz[:, None, cs])
        f1 = _prodpoly(_Q1_A, _Q1, u * u)
        ein = g[:, rs, None] + g[:, None, cs]
        f2 = _prodpoly(_Q2_A, _Q2, ein * ein)
        fcp = fcj[:, rs, None] * fcj[:, None, cs]
        return jnp.sum(f1 * f2 * fcp)

    lo = slice(0, _NL)
    hi = slice(_NL, _N)
    full = (pair_block(lo, lo) + 2.0 * pair_block(hi, lo)
            + pair_block(hi, hi))

    ud = ux * ux + uy * uy + uz * uz
    gd = 2.0 * g
    diag = jnp.sum(_prodpoly(_Q1_A, _Q1, ud * ud)
                   * _prodpoly(_Q2_A, _Q2, gd * gd) * (fcj * fcj))

    angular_part = full - diag

    @pl.when(step == 0)
    def _init():
        out_ref[:, :] = jnp.zeros((1, 1), f32)

    out_ref[:, :] += (radial_part + angular_part) * _SCALE


def kernel(species, positions):
    del species
    post = positions.T.astype(jnp.float32)
    out = pl.pallas_call(
        _aev_kernel,
        grid=(_STEPS,),
        in_specs=[
            pl.BlockSpec((3, _N), lambda i: (0, 0)),
            pl.BlockSpec((_N, 3), lambda i: (0, 0)),
        ],
        out_specs=pl.BlockSpec((1, 1), lambda i: (0, 0)),
        out_shape=jax.ShapeDtypeStruct((1, 1), jnp.float32),
    )(post, positions)
    return out[0, 0]
